# baseline (speedup 1.0000x reference)
import jax
import jax.numpy as jnp
from jax.experimental import pallas as pl


def _copy_body(x_ref, o_ref):
    o_ref[...] = x_ref[...]


def kernel(edge_grad_rm, edge_grad_ins, edge_index_rm, edge_index_ins, k_remove, k_insert):
    K_REMOVE = 1280000
    K_INSERT = 320000
    neg_vals, ind_rm = jax.lax.top_k(-edge_grad_rm, K_REMOVE)
    mask_rm = jnp.ones(edge_grad_rm.shape[0], dtype=bool).at[ind_rm].set(False)
    ir_vals, indices_ir = jax.lax.top_k(edge_grad_ins, K_INSERT)
    ind_rm_ir = jnp.take(edge_index_ins, indices_ir, axis=1)
    kept_grad = edge_grad_rm * mask_rm.astype(edge_grad_rm.dtype)
    kept_grad = pl.pallas_call(
        _copy_body,
        out_shape=jax.ShapeDtypeStruct(kept_grad.shape, kept_grad.dtype),
    )(kept_grad)
    out_vals = jnp.concatenate([kept_grad, ir_vals])
    k_residual = (k_remove - K_REMOVE) + (k_insert - K_INSERT)
    out_vals = out_vals + jnp.asarray(k_residual, dtype=out_vals.dtype)
    return out_vals, mask_rm, ind_rm_ir


# speedup vs baseline: 5.2362x; 5.2362x over previous
import functools

import jax
import jax.numpy as jnp
from jax import lax
from jax.experimental import pallas as pl
from jax.experimental.pallas import tpu as pltpu

_LANES = 128
_BLOCK_ROWS = 400
_NBITS = 4
_NBINS = 1 << _NBITS
_NSTAGES = 32 // _NBITS
_MININT = -(2 ** 31)


def _order_key(x):
    b = lax.bitcast_convert_type(x, jnp.int32)
    return b ^ ((b >> 31) & jnp.int32(0x7FFFFFFF))


def _threshold_body(nblocks, k_select, x_ref, thr_ref, rank_ref, hist, state):
    t = pl.program_id(0)
    b = pl.program_id(1)

    @pl.when(jnp.logical_and(t == 0, b == 0))
    def _init():
        state[0] = jnp.int32(0)
        state[1] = jnp.int32(k_select)
        state[2] = jnp.int32(0)
        state[3] = jnp.int32(28)
        for j in range(_NBINS):
            hist[j] = jnp.int32(0)

    s = _order_key(x_ref[...])
    v = s ^ jnp.int32(_MININT)
    prefix = state[0]
    himask = state[2]
    shift = state[3]
    match = (v & himask) == prefix
    nib = (v >> shift) & jnp.int32(0xF)
    for j in range(_NBINS):
        cnt = jnp.sum(jnp.logical_and(match, nib == j).astype(jnp.int32))
        hist[j] = hist[j] + cnt

    @pl.when(b == nblocks - 1)
    def _finish_stage():
        kp = state[1]
        acc = jnp.int32(0)
        jstar = jnp.int32(0)
        below = jnp.int32(0)
        for j in range(_NBINS):
            c = hist[j]
            hit = jnp.logical_and(acc < kp, acc + c >= kp)
            jstar = jstar + jnp.where(hit, jnp.int32(j), 0)
            below = below + jnp.where(hit, acc, 0)
            acc = acc + c
        new_prefix = prefix | (jstar << shift)
        state[0] = new_prefix
        state[1] = kp - below
        state[2] = (himask >> 4) | jnp.int32(-(2 ** 28))
        state[3] = shift - 4
        for j in range(_NBINS):
            hist[j] = jnp.int32(0)

        @pl.when(t == _NSTAGES - 1)
        def _emit():
            thr_ref[0, 0] = new_prefix ^ jnp.int32(_MININT)
            rank_ref[0, 0] = kp - below


def _mask_body(x_ref, thr_ref, rank_ref, mask_ref, kept_ref, tiecnt):
    i = pl.program_id(0)

    @pl.when(i == 0)
    def _init():
        tiecnt[0] = jnp.int32(0)

    x = x_ref[...]
    s = _order_key(x)
    s_thr = thr_ref[0, 0]
    lt = s < s_thr
    eq = s == s_thr

    eqf = eq.astype(jnp.float32)
    rows, lanes = eqf.shape
    lane_lo = (lax.broadcasted_iota(jnp.int32, (lanes, lanes), 0)
               < lax.broadcasted_iota(jnp.int32, (lanes, lanes), 1))
    pre_lane = jnp.dot(eqf, lane_lo.astype(jnp.float32),
                       preferred_element_type=jnp.float32)
    rowsum = jnp.sum(eqf, axis=1, keepdims=True)
    row_lo = (lax.broadcasted_iota(jnp.int32, (rows, rows), 1)
              < lax.broadcasted_iota(jnp.int32, (rows, rows), 0))
    pre_row = jnp.dot(row_lo.astype(jnp.float32), rowsum,
                      preferred_element_type=jnp.float32)
    pre = tiecnt[0].astype(jnp.float32) + pre_row + pre_lane

    r = rank_ref[0, 0].astype(jnp.float32)
    sel = jnp.logical_or(lt, jnp.logical_and(eq, pre < r))
    keep = jnp.logical_not(sel)
    mask_ref[...] = keep.astype(jnp.int32)
    kept_ref[...] = x * keep.astype(jnp.float32)
    tiecnt[0] = tiecnt[0] + jnp.sum(eq.astype(jnp.int32))


def _remove_select(x, k_select, block_rows=_BLOCK_ROWS, interpret=False):
    n = x.shape[0]
    assert n % _LANES == 0
    rows = n // _LANES
    assert rows % block_rows == 0
    nblocks = rows // block_rows
    x2 = x.reshape(rows, _LANES)

    thr, rank = pl.pallas_call(
        functools.partial(_threshold_body, nblocks, k_select),
        grid=(_NSTAGES, nblocks),
        in_specs=[pl.BlockSpec((block_rows, _LANES), lambda t, b: (b, 0))],
        out_specs=[
            pl.BlockSpec(memory_space=pltpu.SMEM),
            pl.BlockSpec(memory_space=pltpu.SMEM),
        ],
        out_shape=[
            jax.ShapeDtypeStruct((1, 1), jnp.int32),
            jax.ShapeDtypeStruct((1, 1), jnp.int32),
        ],
        scratch_shapes=[
            pltpu.SMEM((_NBINS,), jnp.int32),
            pltpu.SMEM((4,), jnp.int32),
        ],
        interpret=interpret,
    )(x2)

    mask2, kept2 = pl.pallas_call(
        _mask_body,
        grid=(nblocks,),
        in_specs=[
            pl.BlockSpec((block_rows, _LANES), lambda b: (b, 0)),
            pl.BlockSpec(memory_space=pltpu.SMEM),
            pl.BlockSpec(memory_space=pltpu.SMEM),
        ],
        out_specs=[
            pl.BlockSpec((block_rows, _LANES), lambda b: (b, 0)),
            pl.BlockSpec((block_rows, _LANES), lambda b: (b, 0)),
        ],
        out_shape=[
            jax.ShapeDtypeStruct((rows, _LANES), jnp.int32),
            jax.ShapeDtypeStruct((rows, _LANES), jnp.float32),
        ],
        scratch_shapes=[pltpu.SMEM((1,), jnp.int32)],
        interpret=interpret,
    )(x2, thr, rank)
    return mask2.reshape(n), kept2.reshape(n)


def kernel(edge_grad_rm, edge_grad_ins, edge_index_rm, edge_index_ins,
           k_remove, k_insert):
    K_REMOVE = 1280000
    K_INSERT = 320000
    mask_i32, kept_grad = _remove_select(edge_grad_rm, K_REMOVE)
    mask_rm = mask_i32.astype(bool)
    ir_vals, indices_ir = jax.lax.top_k(edge_grad_ins, K_INSERT)
    ind_rm_ir = jnp.take(edge_index_ins, indices_ir, axis=1)

    out_vals = jnp.concatenate([kept_grad, ir_vals])
    k_residual = (k_remove - K_REMOVE) + (k_insert - K_INSERT)
    out_vals = out_vals + jnp.asarray(k_residual, dtype=out_vals.dtype)
    return out_vals, mask_rm, ind_rm_ir
